# baseline (device time: 242647 ns/iter reference)
import functools

import jax
import jax.numpy as jnp
from jax import lax
from jax.experimental import pallas as pl
from jax.experimental.pallas import tpu as pltpu

N_DEV = 4
HALF = 512


def kernel(x, w_mat):
    M, K = x.shape
    _, N = w_mat.shape
    NC = N // N_DEV
    KBLK = 1024
    KB = K // KBLK

    my = lax.axis_index("i")
    order = jnp.mod(my + 1 + jnp.arange(N_DEV, dtype=jnp.int32), N_DEV)

    def body(order_ref, x_ref, w_ref, out_ref, acc_ref, send_ref, recv_ref,
             send_sems, recv_sems, local_sem):
        s = pl.program_id(0)
        kb = pl.program_id(1)
        my_pos = lax.axis_index("i")

        @pl.when(jnp.logical_and(s == 0, kb == 0))
        def _entry_barrier():
            bsem = pltpu.get_barrier_semaphore()
            for d in range(1, N_DEV):
                pl.semaphore_signal(
                    bsem, inc=1,
                    device_id=((my_pos + d) % N_DEV,),
                    device_id_type=pl.DeviceIdType.MESH,
                )
            pl.semaphore_wait(bsem, N_DEV - 1)

        xb = x_ref[...].astype(jnp.bfloat16)
        for h in range(NC // HALF):
            cs = slice(h * HALF, (h + 1) * HALF)
            wb = w_ref[:, cs].astype(jnp.bfloat16)
            part = jnp.dot(xb, wb, preferred_element_type=jnp.float32)

            @pl.when(kb == 0)
            def _(part=part, cs=cs):
                acc_ref[:, cs] = part

            @pl.when(kb != 0)
            def _(part=part, cs=cs):
                acc_ref[:, cs] = acc_ref[:, cs] + part

        @pl.when(kb == KB - 1)
        def _finish_chunk():
            for h in range(NC // HALF):
                cs = slice(h * HALF, (h + 1) * HALF)
                y = acc_ref[:, cs]
                acc_ref[:, cs] = y * jax.nn.sigmoid(y)

            for j in range(N_DEV - 1):
                slot = j % 2

                @pl.when(s == j)
                def _send_remote(j=j, slot=slot):
                    if j == 2:
                        w0 = pltpu.make_async_remote_copy(
                            src_ref=send_ref.at[0],
                            dst_ref=recv_ref.at[0],
                            send_sem=send_sems.at[0],
                            recv_sem=recv_sems.at[0],
                            device_id=(order_ref[0],),
                            device_id_type=pl.DeviceIdType.MESH,
                        )
                        w0.wait_send()
                    for h in range(NC // HALF):
                        cs = slice(h * HALF, (h + 1) * HALF)
                        send_ref[slot, :, cs] = (
                            acc_ref[:, cs].astype(jnp.bfloat16))
                    rdma = pltpu.make_async_remote_copy(
                        src_ref=send_ref.at[slot],
                        dst_ref=recv_ref.at[j],
                        send_sem=send_sems.at[j],
                        recv_sem=recv_sems.at[j],
                        device_id=(order_ref[j],),
                        device_id_type=pl.DeviceIdType.MESH,
                    )
                    rdma.start()

            @pl.when(s == N_DEV - 1)
            def _finish_all():
                cp = pltpu.make_async_copy(
                    acc_ref, out_ref.at[pl.ds(my_pos * M, M), :], local_sem)
                cp.start()
                cp.wait()
                for sj in range(N_DEV - 1):
                    src_dev = (my_pos - 1 - sj) % N_DEV
                    wr = pltpu.make_async_remote_copy(
                        src_ref=send_ref.at[0],
                        dst_ref=recv_ref.at[sj],
                        send_sem=send_sems.at[sj],
                        recv_sem=recv_sems.at[sj],
                        device_id=(my_pos,),
                        device_id_type=pl.DeviceIdType.MESH,
                    )
                    wr.wait_recv()
                    for h in range(NC // HALF):
                        cs = slice(h * HALF, (h + 1) * HALF)
                        acc_ref[:, cs] = recv_ref[sj, :, cs].astype(
                            jnp.float32)
                    cp = pltpu.make_async_copy(
                        acc_ref, out_ref.at[pl.ds(src_dev * M, M), :],
                        local_sem)
                    cp.start()
                    cp.wait()
                for j in (1, 2):
                    wd = pltpu.make_async_remote_copy(
                        src_ref=send_ref.at[j % 2],
                        dst_ref=recv_ref.at[j],
                        send_sem=send_sems.at[j],
                        recv_sem=recv_sems.at[j],
                        device_id=(order_ref[j],),
                        device_id_type=pl.DeviceIdType.MESH,
                    )
                    wd.wait_send()

                @functools.partial(
                    pl.run_scoped, sem2=pltpu.SemaphoreType.REGULAR)
                def _exit_barrier(sem2):
                    for d in range(1, N_DEV):
                        pl.semaphore_signal(
                            sem2, inc=1,
                            device_id=((my_pos + d) % N_DEV,),
                            device_id_type=pl.DeviceIdType.MESH,
                        )
                    pl.semaphore_wait(sem2, N_DEV - 1)

    grid_spec = pltpu.PrefetchScalarGridSpec(
        num_scalar_prefetch=1,
        grid=(N_DEV, KB),
        in_specs=[
            pl.BlockSpec((M, KBLK), lambda s, kb, ord_: (0, kb)),
            pl.BlockSpec((KBLK, NC), lambda s, kb, ord_: (kb, ord_[s])),
        ],
        out_specs=pl.BlockSpec(memory_space=pl.ANY),
        scratch_shapes=[
            pltpu.VMEM((M, NC), jnp.float32),
            pltpu.VMEM((2, M, NC), jnp.bfloat16),
            pltpu.VMEM((N_DEV - 1, M, NC), jnp.bfloat16),
            pltpu.SemaphoreType.DMA((N_DEV - 1,)),
            pltpu.SemaphoreType.DMA((N_DEV - 1,)),
            pltpu.SemaphoreType.DMA,
        ],
    )
    return pl.pallas_call(
        body,
        out_shape=jax.ShapeDtypeStruct((N_DEV * M, NC), jnp.float32),
        grid_spec=grid_spec,
        compiler_params=pltpu.CompilerParams(
            collective_id=0, vmem_limit_bytes=63 * 1024 * 1024),
    )(order, x, w_mat)
